# baseline (speedup 1.0000x reference)
import functools

import jax
import jax.numpy as jnp
from jax import lax
from jax.experimental import pallas as pl
from jax.experimental.pallas import tpu as pltpu
from jax.experimental.pallas import tpu_sc as plsc

N = 10000
E = 320000
F_IN = 128
H = 256
G = 64

N_PAD = 10240
E_PAD = 327680
KC = 160
CH = 128
RPT = N_PAD // 16
BLK = 512
NBLK = N_PAD // BLK


AROWS = 656
BLK_E = 2048
NBLK_E = E_PAD // BLK_E


def _mesh():
    return plsc.VectorSubcoreMesh(core_axis_name="c", subcore_axis_name="s")


def _zero_acc(acc, width):
    @pl.loop(0, AROWS)
    def _(i):
        for k in range(width // 16):
            acc[i, pl.ds(k * 16, 16)] = jnp.zeros((16,), jnp.float32)


def _deg_body(dst1_hbm, dst2_hbm, out_hbm, dstv, acc, sem):
    c = lax.axis_index("c")
    s = lax.axis_index("s")
    lo = s * RPT
    ones16 = jnp.ones((16,), jnp.float32)
    _zero_acc(acc, 16)
    for g, dref in ((0, dst1_hbm), (1, dst2_hbm)):
        @pl.when(c == g)
        def _():
            @pl.loop(0, NBLK_E)
            def _(b):
                pltpu.sync_copy(dref.at[pl.ds(b * BLK_E, BLK_E)],
                                dstv.at[pl.ds(0, BLK_E)])

                @pl.loop(0, BLK_E // 16)
                def _(grp):
                    for k in range(16):
                        dv = dstv[pl.ds(grp * 16 + k, 16)]
                        u = dv[0] - lo

                        @pl.when(jnp.logical_and(u >= 0, u < RPT))
                        def _():
                            acc[u, :] = acc[u, :] + ones16

    pltpu.sync_copy(acc.at[pl.ds(0, RPT)],
                    out_hbm.at[pl.ds(c * N_PAD + s * RPT, RPT)])


def _deg(dst1, dst2):
    k = pl.kernel(
        _deg_body,
        out_type=jax.ShapeDtypeStruct((2 * N_PAD, 16), jnp.float32),
        mesh=_mesh(),
        scratch_types=[
            pltpu.VMEM((BLK_E + 16,), jnp.int32),
            pltpu.VMEM((AROWS, 16), jnp.float32),
            pltpu.SemaphoreType.DMA,
        ],
    )
    return k(dst1, dst2).reshape(2, N_PAD, 16)


def _drain(table, idxb, dlb, gbuf, acc, sem):
    pltpu.async_copy(table.at[idxb], gbuf, sem).wait()

    @pl.loop(0, CH)
    def _(j):
        dv = dlb[pl.ds(j, 16)]
        dl = dv[0]
        for kk in range(8):
            sl = pl.ds(kk * 16, 16)
            acc[dl, sl] = acc[dl, sl] + gbuf[j, sl]


def _scan_phase(table, t_off, src_hbm, dst_hbm, out_off, lo, out_hbm,
                srcv, dstv, idxb, dlb, gbuf, acc, sem):
    io = lax.iota(jnp.int32, 16)
    _zero_acc(acc, CH)

    def group(grp, carry):
        cnt, nrow, ivec, dvec = carry
        for k in range(16):
            dv = dstv[pl.ds(grp * 16 + k, 16)]
            sv = srcv[pl.ds(grp * 16 + k, 16)]
            u = dv[0] - lo
            hit = jnp.logical_and(u >= 0, u < RPT)
            pos = jnp.where(hit, cnt, -1)
            lane = io == pos
            ivec = jnp.where(lane, sv[0] + t_off, ivec)
            dvec = jnp.where(lane, u, dvec)
            cnt = cnt + jnp.where(hit, 1, 0)
            full = cnt == 16

            @pl.when(full)
            def _():
                idxb[pl.ds(nrow * 16, 16)] = ivec
                dlb[pl.ds(nrow * 16, 16)] = dvec

            nrow = nrow + jnp.where(full, 1, 0)
            cnt = jnp.where(full, 0, cnt)
            fire = nrow == 8

            @pl.when(fire)
            def _():
                _drain(table, idxb, dlb, gbuf, acc, sem)

            nrow = jnp.where(fire, 0, nrow)
        return cnt, nrow, ivec, dvec

    def block(b, carry):
        pltpu.sync_copy(src_hbm.at[pl.ds(b * BLK_E, BLK_E)],
                        srcv.at[pl.ds(0, BLK_E)])
        pltpu.sync_copy(dst_hbm.at[pl.ds(b * BLK_E, BLK_E)],
                        dstv.at[pl.ds(0, BLK_E)])
        return pl.loop(0, BLK_E // 16, init_carry=carry)(group)

    z16 = jnp.zeros((16,), jnp.int32)
    cnt, nrow, ivec, dvec = pl.loop(
        0, NBLK_E, init_carry=(jnp.int32(0), jnp.int32(0), z16, z16))(block)

    ivec = jnp.where(io < cnt, ivec, t_off)
    dvec = jnp.where(io < cnt, dvec, 640)

    @pl.when(cnt > 0)
    def _():
        idxb[pl.ds(nrow * 16, 16)] = ivec
        dlb[pl.ds(nrow * 16, 16)] = dvec

    nrow = nrow + jnp.where(cnt > 0, 1, 0)

    @pl.loop(0, 8)
    def _(r):
        @pl.when(r >= nrow)
        def _():
            idxb[pl.ds(r * 16, 16)] = jnp.zeros((16,), jnp.int32) + t_off
            dlb[pl.ds(r * 16, 16)] = jnp.zeros((16,), jnp.int32) + 640

    _drain(table, idxb, dlb, gbuf, acc, sem)
    pltpu.sync_copy(acc.at[pl.ds(0, RPT)], out_hbm.at[pl.ds(out_off, RPT)])


_PROP_SCRATCH = [
    pltpu.VMEM((BLK_E + 16,), jnp.int32),
    pltpu.VMEM((BLK_E + 16,), jnp.int32),
    pltpu.VMEM((CH,), jnp.int32),
    pltpu.VMEM((CH + 16,), jnp.int32),
    pltpu.VMEM((CH, CH), jnp.float32),
    pltpu.VMEM((AROWS, CH), jnp.float32),
    pltpu.SemaphoreType.DMA,
]


def _prop1_body(t_hbm, src1, dst1, src2, dst2, out_hbm,
                srcv, dstv, idxb, dlb, gbuf, acc, sem):
    c = lax.axis_index("c")
    s = lax.axis_index("s")
    for g, (sr, dr) in ((0, (src1, dst1)), (1, (src2, dst2))):
        @pl.when(c == g)
        def _():
            _scan_phase(t_hbm, c * N_PAD, sr, dr, c * N_PAD + s * RPT,
                        s * RPT, out_hbm, srcv, dstv, idxb, dlb, gbuf, acc, sem)


def _prop23_body(t1_hbm, t2_hbm, src1, dst1, src2, dst2, out_hbm,
                 srcv, dstv, idxb, dlb, gbuf, acc, sem):
    c = lax.axis_index("c")
    s = lax.axis_index("s")
    for g, (tab, sr, dr) in ((0, (t1_hbm, src1, dst1)),
                             (1, (t2_hbm, src2, dst2))):
        _scan_phase(tab, c * N_PAD, sr, dr, (g * 2 + c) * N_PAD + s * RPT,
                    s * RPT, out_hbm, srcv, dstv, idxb, dlb, gbuf, acc, sem)


def _prop1(t_flat, src1, dst1, src2, dst2):
    k = pl.kernel(
        _prop1_body,
        out_type=jax.ShapeDtypeStruct((2 * N_PAD, CH), jnp.float32),
        mesh=_mesh(),
        scratch_types=_PROP_SCRATCH,
    )
    return k(t_flat, src1, dst1, src2, dst2).reshape(2, N_PAD, CH)


def _prop23(t1_flat, t2_flat, src1, dst1, src2, dst2):
    k = pl.kernel(
        _prop23_body,
        out_type=jax.ShapeDtypeStruct((4 * N_PAD, CH), jnp.float32),
        mesh=_mesh(),
        scratch_types=_PROP_SCRATCH,
    )
    return k(t1_flat, t2_flat, src1, dst1, src2, dst2).reshape(2, 2, N_PAD, CH)


def _pre_body(x_ref, dacc_ref, t_ref, dinv_ref):
    deg = dacc_ref[0, :, 0:1] + 1.0
    dv = lax.rsqrt(deg)
    dvb = jnp.broadcast_to(dv, (BLK, CH))
    dinv_ref[0] = dvb
    t_ref[0] = x_ref[0] * dvb


def _pre(x_pad, degacc):
    return pl.pallas_call(
        _pre_body,
        grid=(2, NBLK),
        in_specs=[
            pl.BlockSpec((1, BLK, CH), lambda g, i: (g, i, 0)),
            pl.BlockSpec((1, BLK, 16), lambda g, i: (g, i, 0)),
        ],
        out_specs=[
            pl.BlockSpec((1, BLK, CH), lambda g, i: (g, i, 0)),
            pl.BlockSpec((1, BLK, CH), lambda g, i: (g, i, 0)),
        ],
        out_shape=[
            jax.ShapeDtypeStruct((2, N_PAD, CH), jnp.float32),
            jax.ShapeDtypeStruct((2, N_PAD, CH), jnp.float32),
        ],
    )(x_pad, degacc)


def _layer_body(hin, scale_out, s_ref, t_ref, dinv_ref, w_ref, b_ref, out_ref):
    i = pl.program_id(1)
    dv = dinv_ref[0]
    acc = jnp.broadcast_to(b_ref[...], (BLK, 2 * CH))
    for hh in range(hin):
        u = (s_ref[0, hh] + t_ref[0, hh]) * dv
        acc = acc + jnp.dot(u, w_ref[hh], preferred_element_type=jnp.float32)
    rows = i * BLK + lax.broadcasted_iota(jnp.int32, (BLK, 2 * CH), 0)
    x = jnp.where(rows < N, jnp.maximum(acc, 0.0), 0.0)
    if scale_out:
        out_ref[0, 0] = x[:, :CH] * dv
        out_ref[0, 1] = x[:, CH:] * dv
    else:
        out_ref[0, 0] = x[:, :CH]
        out_ref[0, 1] = x[:, CH:]


def _layer(S, T, dinv, W, b, scale_out):
    hin = W.shape[0]
    return pl.pallas_call(
        functools.partial(_layer_body, hin, scale_out),
        grid=(2, NBLK),
        in_specs=[
            pl.BlockSpec((1, hin, BLK, CH), lambda g, i: (g, 0, i, 0)),
            pl.BlockSpec((1, hin, BLK, CH), lambda g, i: (g, 0, i, 0)),
            pl.BlockSpec((1, BLK, CH), lambda g, i: (g, i, 0)),
            pl.BlockSpec((hin, CH, 2 * CH), lambda g, i: (0, 0, 0)),
            pl.BlockSpec((1, 2 * CH), lambda g, i: (0, 0)),
        ],
        out_specs=pl.BlockSpec((1, 2, BLK, CH), lambda g, i: (g, 0, i, 0)),
        out_shape=jax.ShapeDtypeStruct((2, 2, N_PAD, CH), jnp.float32),
    )(S, T, dinv, W, b)


def _head_body(x1_ref, x2_ref, b1_ref, b2_ref,
               c1w, c1b, c2w, c2b, c3w, c3b, c4w, c4b, out_ref,
               z1a, z2a, cnt1, cnt2):
    i = pl.program_id(0)

    @pl.when(i == 0)
    def _():
        z1a[...] = jnp.zeros((G, 2 * CH), jnp.float32)
        z2a[...] = jnp.zeros((G, 2 * CH), jnp.float32)
        cnt1[...] = jnp.zeros((G, CH), jnp.float32)
        cnt2[...] = jnp.zeros((G, CH), jnp.float32)

    gid = lax.broadcasted_iota(jnp.int32, (G, BLK), 0)
    for x_ref, b_ref, za, ca in ((x1_ref, b1_ref, z1a, cnt1),
                                 (x2_ref, b2_ref, z2a, cnt2)):
        m = (jnp.broadcast_to(b_ref[...], (G, BLK)) == gid).astype(jnp.float32)
        h0 = jnp.dot(m, x_ref[0], preferred_element_type=jnp.float32)
        h1 = jnp.dot(m, x_ref[1], preferred_element_type=jnp.float32)
        za[...] += jnp.concatenate([h0, h1], axis=1)
        ca[...] += jnp.broadcast_to(
            jnp.sum(m, axis=1, keepdims=True), (G, CH))

    @pl.when(i == pl.num_programs(0) - 1)
    def _():
        z1 = z1a[...] / jnp.maximum(cnt1[...][:, 0:1], 1.0)
        z2 = z2a[...] / jnp.maximum(cnt2[...][:, 0:1], 1.0)
        z = jnp.concatenate([z1, z2], axis=1)
        z = jnp.maximum(jnp.dot(z, c1w[...], preferred_element_type=jnp.float32)
                        + c1b[...], 0.0)
        z = jnp.maximum(jnp.dot(z, c2w[...], preferred_element_type=jnp.float32)
                        + c2b[...], 0.0)
        z = jnp.maximum(jnp.dot(z, c3w[...], preferred_element_type=jnp.float32)
                        + c3b[...], 0.0)
        z = jnp.dot(z, c4w[...], preferred_element_type=jnp.float32) + c4b[...]
        out_ref[...] = jax.nn.sigmoid(z)


def _head(X3_1, X3_2, batch1, batch2, C1w, C1b, C2w, C2b, C3w, C3b, C4w, C4b):
    const2 = lambda a, b: pl.BlockSpec((a, b), lambda i: (0, 0))
    return pl.pallas_call(
        _head_body,
        grid=(NBLK,),
        in_specs=[
            pl.BlockSpec((2, BLK, CH), lambda i: (0, i, 0)),
            pl.BlockSpec((2, BLK, CH), lambda i: (0, i, 0)),
            pl.BlockSpec((1, BLK), lambda i: (0, i)),
            pl.BlockSpec((1, BLK), lambda i: (0, i)),
            const2(2 * H, 512), const2(1, 512),
            const2(512, 256), const2(1, 256),
            const2(256, 128), const2(1, 128),
            const2(128, 64), const2(1, 64),
        ],
        out_specs=pl.BlockSpec((G, G), lambda i: (0, 0)),
        out_shape=jax.ShapeDtypeStruct((G, G), jnp.float32),
        scratch_shapes=[
            pltpu.VMEM((G, 2 * CH), jnp.float32),
            pltpu.VMEM((G, 2 * CH), jnp.float32),
            pltpu.VMEM((G, CH), jnp.float32),
            pltpu.VMEM((G, CH), jnp.float32),
        ],
    )(X3_1, X3_2, batch1, batch2, C1w, C1b, C2w, C2b, C3w, C3b, C4w, C4b)


def _pad_edges(ei):
    ei = ei.astype(jnp.int32)
    src = jnp.concatenate([ei[0], jnp.zeros((E_PAD - E,), jnp.int32)])
    dst = jnp.concatenate([ei[1], jnp.full((E_PAD - E,), N_PAD, jnp.int32)])
    return src, dst


def kernel(x_1, edge_index_1, x_1_batch, x_2, edge_index_2, x_2_batch,
           W1, b1, W2, b2, W3, b3, C1w, C1b, C2w, C2b, C3w, C3b, C4w, C4b):
    src1, dst1 = _pad_edges(edge_index_1)
    src2, dst2 = _pad_edges(edge_index_2)

    x_pad = jnp.stack([
        jnp.pad(x_1, ((0, N_PAD - N), (0, 0))),
        jnp.pad(x_2, ((0, N_PAD - N), (0, 0))),
    ])

    degacc = _deg(dst1, dst2)
    T0, dinv = _pre(x_pad, degacc)

    S1 = _prop1(T0.reshape(2 * N_PAD, CH), src1, dst1, src2, dst2)
    T1 = _layer(S1.reshape(2, 1, N_PAD, CH), T0.reshape(2, 1, N_PAD, CH),
                dinv, W1.reshape(1, CH, 2 * CH), b1.reshape(1, 2 * CH), True)

    S2 = _prop23(T1[0].reshape(2 * N_PAD, CH), T1[1].reshape(2 * N_PAD, CH),
                 src1, dst1, src2, dst2)
    T2 = _layer(S2, T1, dinv, W2.reshape(2, CH, 2 * CH),
                b2.reshape(1, 2 * CH), True)

    S3 = _prop23(T2[0].reshape(2 * N_PAD, CH), T2[1].reshape(2 * N_PAD, CH),
                 src1, dst1, src2, dst2)
    X3 = _layer(S3, T2, dinv, W3.reshape(2, CH, 2 * CH),
                b3.reshape(1, 2 * CH), False)

    fillb = jnp.full((N_PAD - N,), G, jnp.int32)
    batch1 = jnp.concatenate([x_1_batch.astype(jnp.int32), fillb]).reshape(1, N_PAD)
    batch2 = jnp.concatenate([x_2_batch.astype(jnp.int32), fillb]).reshape(1, N_PAD)

    return _head(X3[0], X3[1], batch1, batch2,
                 C1w, C1b.reshape(1, -1), C2w, C2b.reshape(1, -1),
                 C3w, C3b.reshape(1, -1), C4w, C4b.reshape(1, -1))


# speedup vs baseline: 7.0251x; 7.0251x over previous
import functools

import jax
import jax.numpy as jnp
from jax import lax
from jax.experimental import pallas as pl
from jax.experimental.pallas import tpu as pltpu
from jax.experimental.pallas import tpu_sc as plsc

N = 10000
E = 320000
F_IN = 128
H = 256
G = 64

N_PAD = 10240
E_PAD = 327680
KC = 160
CH = 128
RPT = N_PAD // 16
BLK = 512
NBLK = N_PAD // BLK


AROWS = 656
BLK_E = 2048
NBLK_E = E_PAD // BLK_E
LCAP = E_PAD + CH


def _mesh():
    return plsc.VectorSubcoreMesh(core_axis_name="c", subcore_axis_name="s")


def _zero_acc(acc, width):
    @pl.loop(0, AROWS)
    def _(i):
        for k in range(width // 16):
            acc[i, pl.ds(k * 16, 16)] = jnp.zeros((16,), jnp.float32)


TRASH_W = 640 << 14


def _deg_body(w1_hbm, w2_hbm, deg_hbm, list_hbm, cnt_hbm,
              wv, ivb, wout, acc, sem):
    c = lax.axis_index("c")
    s = lax.axis_index("s")
    lo = s * RPT
    base = (c * 16 + s) * LCAP
    io = lax.iota(jnp.int32, 16)
    ones16 = jnp.ones((16,), jnp.float32)
    _zero_acc(acc, 16)

    def group(grp, carry):
        cnt, nrow, nb, ivec = carry
        for k in range(16):
            v = wv[pl.ds(grp * 16 + k, 16)]
            w0 = v[0]
            u = (w0 >> 14) - lo
            hit = jnp.logical_and(u >= 0, u < RPT)

            @pl.when(hit)
            def _():
                acc[u, :] = acc[u, :] + ones16

            wl = (w0 & 16383) | (u << 14)
            pos = jnp.where(hit, cnt, -1)
            ivec = jnp.where(io == pos, wl, ivec)
            cnt = cnt + jnp.where(hit, 1, 0)
            full = cnt == 16

            @pl.when(full)
            def _():
                wout[pl.ds(nrow * 16, 16)] = ivec

            nrow = nrow + jnp.where(full, 1, 0)
            cnt = jnp.where(full, 0, cnt)
            fire = nrow == 8

            @pl.when(fire)
            def _():
                pltpu.sync_copy(wout, list_hbm.at[pl.ds(base + nb * CH, CH)])

            nb = nb + jnp.where(fire, 1, 0)
            nrow = jnp.where(fire, 0, nrow)
        return cnt, nrow, nb, ivec

    def mk_block(wref):
        def block(b, carry):
            pltpu.sync_copy(wref.at[pl.ds(b * BLK_E, BLK_E)],
                            wv.at[pl.ds(0, BLK_E)])
            return pl.loop(0, BLK_E // 16, init_carry=carry)(group)
        return block

    def scan(wref):
        carry0 = (jnp.int32(0), jnp.int32(0), jnp.int32(0),
                  jnp.zeros((16,), jnp.int32))
        cnt, nrow, nb, ivec = pl.loop(0, NBLK_E, init_carry=carry0)(
            mk_block(wref))

        ivec = jnp.where(io < cnt, ivec, TRASH_W)

        @pl.when(cnt > 0)
        def _():
            wout[pl.ds(nrow * 16, 16)] = ivec

        nrow = nrow + jnp.where(cnt > 0, 1, 0)

        @pl.loop(0, 8)
        def _(r):
            @pl.when(r >= nrow)
            def _():
                wout[pl.ds(r * 16, 16)] = jnp.zeros((16,), jnp.int32) + TRASH_W

        pltpu.sync_copy(wout, list_hbm.at[pl.ds(base + nb * CH, CH)])
        ivb[...] = jnp.zeros((16,), jnp.int32) + (nb + 1)
        pltpu.sync_copy(ivb, cnt_hbm.at[pl.ds((c * 16 + s) * 16, 16)])

    for g, wref in ((0, w1_hbm), (1, w2_hbm)):
        @pl.when(c == g)
        def _():
            scan(wref)

    pltpu.sync_copy(acc.at[pl.ds(0, RPT)],
                    deg_hbm.at[pl.ds(c * N_PAD + s * RPT, RPT)])


def _deg(w1, w2):
    k = pl.kernel(
        _deg_body,
        out_type=(
            jax.ShapeDtypeStruct((2 * N_PAD, 16), jnp.float32),
            jax.ShapeDtypeStruct((32 * LCAP,), jnp.int32),
            jax.ShapeDtypeStruct((512,), jnp.int32),
        ),
        mesh=_mesh(),
        scratch_types=[
            pltpu.VMEM((BLK_E + 16,), jnp.int32),
            pltpu.VMEM((16,), jnp.int32),
            pltpu.VMEM((CH,), jnp.int32),
            pltpu.VMEM((AROWS, 16), jnp.float32),
            pltpu.SemaphoreType.DMA,
        ],
    )
    deg, lists, counts = k(w1, w2)
    return deg.reshape(2, N_PAD, 16), lists, counts


def _drain(table, idxb, dlb, gbuf, acc, sem):
    pltpu.async_copy(table.at[idxb], gbuf, sem).wait()

    @pl.loop(0, CH)
    def _(j):
        dv = dlb[pl.ds(j, 16)]
        dl = dv[0]
        for kk in range(8):
            sl = pl.ds(kk * 16, 16)
            acc[dl, sl] = acc[dl, sl] + gbuf[j, sl]


def _consume_phase(table, t_off, lid, list_hbm, cnt_hbm, out_off, out_hbm,
                   cv, wbuf, idxb, dlb, gbuf, acc, sem):
    _zero_acc(acc, CH)
    pltpu.sync_copy(cnt_hbm.at[pl.ds(lid * 16, 16)], cv)
    nb = cv[pl.ds(0, 16)][0]
    base = lid * LCAP

    @pl.loop(0, nb)
    def _(b):
        pltpu.sync_copy(list_hbm.at[pl.ds(base + b * CH, CH)], wbuf)
        for k in range(8):
            v = wbuf[pl.ds(16 * k, 16)]
            idxb[pl.ds(16 * k, 16)] = (v & 16383) + t_off
            dlb[pl.ds(16 * k, 16)] = v >> 14
        _drain(table, idxb, dlb, gbuf, acc, sem)

    pltpu.sync_copy(acc.at[pl.ds(0, RPT)], out_hbm.at[pl.ds(out_off, RPT)])


_PROP_SCRATCH = [
    pltpu.VMEM((16,), jnp.int32),
    pltpu.VMEM((CH,), jnp.int32),
    pltpu.VMEM((CH,), jnp.int32),
    pltpu.VMEM((CH + 16,), jnp.int32),
    pltpu.VMEM((CH, CH), jnp.float32),
    pltpu.VMEM((AROWS, CH), jnp.float32),
    pltpu.SemaphoreType.DMA,
]


def _prop1_body(t_hbm, list_hbm, cnt_hbm, out_hbm,
                cv, wbuf, idxb, dlb, gbuf, acc, sem):
    c = lax.axis_index("c")
    s = lax.axis_index("s")
    _consume_phase(t_hbm, c * N_PAD, c * 16 + s, list_hbm, cnt_hbm,
                   c * N_PAD + s * RPT, out_hbm,
                   cv, wbuf, idxb, dlb, gbuf, acc, sem)


def _prop23_body(t1_hbm, t2_hbm, list_hbm, cnt_hbm, out_hbm,
                 cv, wbuf, idxb, dlb, gbuf, acc, sem):
    c = lax.axis_index("c")
    s = lax.axis_index("s")
    for g, tab in ((0, t1_hbm), (1, t2_hbm)):
        _consume_phase(tab, c * N_PAD, g * 16 + s, list_hbm, cnt_hbm,
                       (g * 2 + c) * N_PAD + s * RPT, out_hbm,
                       cv, wbuf, idxb, dlb, gbuf, acc, sem)


def _prop1(t_flat, lists, counts):
    k = pl.kernel(
        _prop1_body,
        out_type=jax.ShapeDtypeStruct((2 * N_PAD, CH), jnp.float32),
        mesh=_mesh(),
        scratch_types=_PROP_SCRATCH,
    )
    return k(t_flat, lists, counts).reshape(2, N_PAD, CH)


def _prop23(t1_flat, t2_flat, lists, counts):
    k = pl.kernel(
        _prop23_body,
        out_type=jax.ShapeDtypeStruct((4 * N_PAD, CH), jnp.float32),
        mesh=_mesh(),
        scratch_types=_PROP_SCRATCH,
    )
    return k(t1_flat, t2_flat, lists, counts).reshape(2, 2, N_PAD, CH)


def _pre_body(x_ref, dacc_ref, t_ref, dinv_ref):
    deg = dacc_ref[0, :, 0:1] + 1.0
    dv = lax.rsqrt(deg)
    dvb = jnp.broadcast_to(dv, (BLK, CH))
    dinv_ref[0] = dvb
    t_ref[0] = x_ref[0] * dvb


def _pre(x_pad, degacc):
    return pl.pallas_call(
        _pre_body,
        grid=(2, NBLK),
        in_specs=[
            pl.BlockSpec((1, BLK, CH), lambda g, i: (g, i, 0)),
            pl.BlockSpec((1, BLK, 16), lambda g, i: (g, i, 0)),
        ],
        out_specs=[
            pl.BlockSpec((1, BLK, CH), lambda g, i: (g, i, 0)),
            pl.BlockSpec((1, BLK, CH), lambda g, i: (g, i, 0)),
        ],
        out_shape=[
            jax.ShapeDtypeStruct((2, N_PAD, CH), jnp.float32),
            jax.ShapeDtypeStruct((2, N_PAD, CH), jnp.float32),
        ],
    )(x_pad, degacc)


def _layer_body(hin, scale_out, s_ref, t_ref, dinv_ref, w_ref, b_ref, out_ref):
    i = pl.program_id(1)
    dv = dinv_ref[0]
    acc = jnp.broadcast_to(b_ref[...], (BLK, 2 * CH))
    for hh in range(hin):
        u = (s_ref[0, hh] + t_ref[0, hh]) * dv
        acc = acc + jnp.dot(u, w_ref[hh], preferred_element_type=jnp.float32)
    rows = i * BLK + lax.broadcasted_iota(jnp.int32, (BLK, 2 * CH), 0)
    x = jnp.where(rows < N, jnp.maximum(acc, 0.0), 0.0)
    if scale_out:
        out_ref[0, 0] = x[:, :CH] * dv
        out_ref[0, 1] = x[:, CH:] * dv
    else:
        out_ref[0, 0] = x[:, :CH]
        out_ref[0, 1] = x[:, CH:]


def _layer(S, T, dinv, W, b, scale_out):
    hin = W.shape[0]
    return pl.pallas_call(
        functools.partial(_layer_body, hin, scale_out),
        grid=(2, NBLK),
        in_specs=[
            pl.BlockSpec((1, hin, BLK, CH), lambda g, i: (g, 0, i, 0)),
            pl.BlockSpec((1, hin, BLK, CH), lambda g, i: (g, 0, i, 0)),
            pl.BlockSpec((1, BLK, CH), lambda g, i: (g, i, 0)),
            pl.BlockSpec((hin, CH, 2 * CH), lambda g, i: (0, 0, 0)),
            pl.BlockSpec((1, 2 * CH), lambda g, i: (0, 0)),
        ],
        out_specs=pl.BlockSpec((1, 2, BLK, CH), lambda g, i: (g, 0, i, 0)),
        out_shape=jax.ShapeDtypeStruct((2, 2, N_PAD, CH), jnp.float32),
    )(S, T, dinv, W, b)


def _head_body(x1_ref, x2_ref, b1_ref, b2_ref,
               c1w, c1b, c2w, c2b, c3w, c3b, c4w, c4b, out_ref,
               z1a, z2a, cnt1, cnt2):
    i = pl.program_id(0)

    @pl.when(i == 0)
    def _():
        z1a[...] = jnp.zeros((G, 2 * CH), jnp.float32)
        z2a[...] = jnp.zeros((G, 2 * CH), jnp.float32)
        cnt1[...] = jnp.zeros((G, CH), jnp.float32)
        cnt2[...] = jnp.zeros((G, CH), jnp.float32)

    gid = lax.broadcasted_iota(jnp.int32, (G, BLK), 0)
    for x_ref, b_ref, za, ca in ((x1_ref, b1_ref, z1a, cnt1),
                                 (x2_ref, b2_ref, z2a, cnt2)):
        m = (jnp.broadcast_to(b_ref[...], (G, BLK)) == gid).astype(jnp.float32)
        h0 = jnp.dot(m, x_ref[0], preferred_element_type=jnp.float32)
        h1 = jnp.dot(m, x_ref[1], preferred_element_type=jnp.float32)
        za[...] += jnp.concatenate([h0, h1], axis=1)
        ca[...] += jnp.broadcast_to(
            jnp.sum(m, axis=1, keepdims=True), (G, CH))

    @pl.when(i == pl.num_programs(0) - 1)
    def _():
        z1 = z1a[...] / jnp.maximum(cnt1[...][:, 0:1], 1.0)
        z2 = z2a[...] / jnp.maximum(cnt2[...][:, 0:1], 1.0)
        z = jnp.concatenate([z1, z2], axis=1)
        z = jnp.maximum(jnp.dot(z, c1w[...], preferred_element_type=jnp.float32)
                        + c1b[...], 0.0)
        z = jnp.maximum(jnp.dot(z, c2w[...], preferred_element_type=jnp.float32)
                        + c2b[...], 0.0)
        z = jnp.maximum(jnp.dot(z, c3w[...], preferred_element_type=jnp.float32)
                        + c3b[...], 0.0)
        z = jnp.dot(z, c4w[...], preferred_element_type=jnp.float32) + c4b[...]
        out_ref[...] = jax.nn.sigmoid(z)


def _head(X3_1, X3_2, batch1, batch2, C1w, C1b, C2w, C2b, C3w, C3b, C4w, C4b):
    const2 = lambda a, b: pl.BlockSpec((a, b), lambda i: (0, 0))
    return pl.pallas_call(
        _head_body,
        grid=(NBLK,),
        in_specs=[
            pl.BlockSpec((2, BLK, CH), lambda i: (0, i, 0)),
            pl.BlockSpec((2, BLK, CH), lambda i: (0, i, 0)),
            pl.BlockSpec((1, BLK), lambda i: (0, i)),
            pl.BlockSpec((1, BLK), lambda i: (0, i)),
            const2(2 * H, 512), const2(1, 512),
            const2(512, 256), const2(1, 256),
            const2(256, 128), const2(1, 128),
            const2(128, 64), const2(1, 64),
        ],
        out_specs=pl.BlockSpec((G, G), lambda i: (0, 0)),
        out_shape=jax.ShapeDtypeStruct((G, G), jnp.float32),
        scratch_shapes=[
            pltpu.VMEM((G, 2 * CH), jnp.float32),
            pltpu.VMEM((G, 2 * CH), jnp.float32),
            pltpu.VMEM((G, CH), jnp.float32),
            pltpu.VMEM((G, CH), jnp.float32),
        ],
    )(X3_1, X3_2, batch1, batch2, C1w, C1b, C2w, C2b, C3w, C3b, C4w, C4b)


def _pack_edges(ei):
    ei = ei.astype(jnp.int32)
    w = ei[0] | (ei[1] << 14)
    return jnp.concatenate([w, jnp.full((E_PAD - E,), N_PAD << 14, jnp.int32)])


def kernel(x_1, edge_index_1, x_1_batch, x_2, edge_index_2, x_2_batch,
           W1, b1, W2, b2, W3, b3, C1w, C1b, C2w, C2b, C3w, C3b, C4w, C4b):
    w1 = _pack_edges(edge_index_1)
    w2 = _pack_edges(edge_index_2)

    x_pad = jnp.stack([
        jnp.pad(x_1, ((0, N_PAD - N), (0, 0))),
        jnp.pad(x_2, ((0, N_PAD - N), (0, 0))),
    ])

    degacc, lists, counts = _deg(w1, w2)
    T0, dinv = _pre(x_pad, degacc)

    S1 = _prop1(T0.reshape(2 * N_PAD, CH), lists, counts)
    T1 = _layer(S1.reshape(2, 1, N_PAD, CH), T0.reshape(2, 1, N_PAD, CH),
                dinv, W1.reshape(1, CH, 2 * CH), b1.reshape(1, 2 * CH), True)

    S2 = _prop23(T1[0].reshape(2 * N_PAD, CH), T1[1].reshape(2 * N_PAD, CH),
                 lists, counts)
    T2 = _layer(S2, T1, dinv, W2.reshape(2, CH, 2 * CH),
                b2.reshape(1, 2 * CH), True)

    S3 = _prop23(T2[0].reshape(2 * N_PAD, CH), T2[1].reshape(2 * N_PAD, CH),
                 lists, counts)
    X3 = _layer(S3, T2, dinv, W3.reshape(2, CH, 2 * CH),
                b3.reshape(1, 2 * CH), False)

    fillb = jnp.full((N_PAD - N,), G, jnp.int32)
    batch1 = jnp.concatenate([x_1_batch.astype(jnp.int32), fillb]).reshape(1, N_PAD)
    batch2 = jnp.concatenate([x_2_batch.astype(jnp.int32), fillb]).reshape(1, N_PAD)

    return _head(X3[0], X3[1], batch1, batch2,
                 C1w, C1b.reshape(1, -1), C2w, C2b.reshape(1, -1),
                 C3w, C3b.reshape(1, -1), C4w, C4b.reshape(1, -1))


# speedup vs baseline: 7.5159x; 1.0699x over previous
import functools

import jax
import jax.numpy as jnp
from jax import lax
from jax.experimental import pallas as pl
from jax.experimental.pallas import tpu as pltpu
from jax.experimental.pallas import tpu_sc as plsc

N = 10000
E = 320000
F_IN = 128
H = 256
G = 64

N_PAD = 10240
E_PAD = 327680
KC = 160
CH = 128
RPT = N_PAD // 16
BLK = 512
NBLK = N_PAD // BLK


AROWS = 656
BLK_E = 2048
NBLK_E = E_PAD // BLK_E
LCAP = E_PAD + CH


def _mesh():
    return plsc.VectorSubcoreMesh(core_axis_name="c", subcore_axis_name="s")


def _zero_acc(acc, width):
    @pl.loop(0, AROWS)
    def _(i):
        for k in range(width // 16):
            acc[i, pl.ds(k * 16, 16)] = jnp.zeros((16,), jnp.float32)


TRASH_W = 640 << 14


def _deg_body(w1_hbm, w2_hbm, deg_hbm, list_hbm, cnt_hbm,
              wv, ivb, wout, acc, sem):
    c = lax.axis_index("c")
    s = lax.axis_index("s")
    lo = s * RPT
    base = (c * 16 + s) * LCAP
    io = lax.iota(jnp.int32, 16)
    ones16 = jnp.ones((16,), jnp.float32)
    _zero_acc(acc, 16)

    def group(grp, carry):
        cnt, nrow, nb, iv0, iv1 = carry
        for k in range(16):
            v = wv[pl.ds(grp * 16 + k, 16)]
            w0 = v[0]
            u = (w0 >> 14) - lo
            hit = jnp.logical_and(u >= 0, u < RPT)
            us = jnp.where(hit, u, 640)
            acc[us, :] = acc[us, :] + ones16
            wl = (w0 & 16383) | (u << 14)
            pos = jnp.where(hit, cnt, -1)
            iv0 = jnp.where(io == pos, wl, iv0)
            iv1 = jnp.where(io == pos - 16, wl, iv1)
            cnt = cnt + jnp.where(hit, 1, 0)

        spill = cnt >= 16

        @pl.when(spill)
        def _():
            wout[pl.ds(nrow * 16, 16)] = iv0

        nrow = nrow + jnp.where(spill, 1, 0)
        cnt = jnp.where(spill, cnt - 16, cnt)
        iv0 = jnp.where(spill, iv1, iv0)
        fire = nrow == 8

        @pl.when(fire)
        def _():
            pltpu.sync_copy(wout, list_hbm.at[pl.ds(base + nb * CH, CH)])

        nb = nb + jnp.where(fire, 1, 0)
        nrow = jnp.where(fire, 0, nrow)
        return cnt, nrow, nb, iv0, iv1

    def mk_block(wref):
        def block(b, carry):
            pltpu.sync_copy(wref.at[pl.ds(b * BLK_E, BLK_E)],
                            wv.at[pl.ds(0, BLK_E)])
            return pl.loop(0, BLK_E // 16, init_carry=carry)(group)
        return block

    def scan(wref):
        carry0 = (jnp.int32(0), jnp.int32(0), jnp.int32(0),
                  jnp.zeros((16,), jnp.int32), jnp.zeros((16,), jnp.int32))
        cnt, nrow, nb, ivec, _iv1 = pl.loop(0, NBLK_E, init_carry=carry0)(
            mk_block(wref))

        ivec = jnp.where(io < cnt, ivec, TRASH_W)

        @pl.when(cnt > 0)
        def _():
            wout[pl.ds(nrow * 16, 16)] = ivec

        nrow = nrow + jnp.where(cnt > 0, 1, 0)

        @pl.loop(0, 8)
        def _(r):
            @pl.when(r >= nrow)
            def _():
                wout[pl.ds(r * 16, 16)] = jnp.zeros((16,), jnp.int32) + TRASH_W

        pltpu.sync_copy(wout, list_hbm.at[pl.ds(base + nb * CH, CH)])
        ivb[...] = jnp.zeros((16,), jnp.int32) + (nb + 1)
        pltpu.sync_copy(ivb, cnt_hbm.at[pl.ds((c * 16 + s) * 16, 16)])

    for g, wref in ((0, w1_hbm), (1, w2_hbm)):
        @pl.when(c == g)
        def _():
            scan(wref)

    pltpu.sync_copy(acc.at[pl.ds(0, RPT)],
                    deg_hbm.at[pl.ds(c * N_PAD + s * RPT, RPT)])


def _deg(w1, w2):
    k = pl.kernel(
        _deg_body,
        out_type=(
            jax.ShapeDtypeStruct((2 * N_PAD, 16), jnp.float32),
            jax.ShapeDtypeStruct((32 * LCAP,), jnp.int32),
            jax.ShapeDtypeStruct((512,), jnp.int32),
        ),
        mesh=_mesh(),
        scratch_types=[
            pltpu.VMEM((BLK_E + 16,), jnp.int32),
            pltpu.VMEM((16,), jnp.int32),
            pltpu.VMEM((CH,), jnp.int32),
            pltpu.VMEM((AROWS, 16), jnp.float32),
            pltpu.SemaphoreType.DMA,
        ],
    )
    deg, lists, counts = k(w1, w2)
    return deg.reshape(2, N_PAD, 16), lists, counts


def _drain(table, idxb, dlb, gbuf, acc, sem):
    pltpu.async_copy(table.at[idxb], gbuf, sem).wait()

    @pl.loop(0, CH)
    def _(j):
        dv = dlb[pl.ds(j, 16)]
        dl = dv[0]
        for kk in range(8):
            sl = pl.ds(kk * 16, 16)
            acc[dl, sl] = acc[dl, sl] + gbuf[j, sl]


def _consume_phase(table, t_off, lid, list_hbm, cnt_hbm, out_off, out_hbm,
                   cv, wbuf, idxb, dlb, gbuf, acc, sem):
    _zero_acc(acc, CH)
    pltpu.sync_copy(cnt_hbm.at[pl.ds(lid * 16, 16)], cv)
    nb = cv[pl.ds(0, 16)][0]
    base = lid * LCAP

    @pl.loop(0, nb)
    def _(b):
        pltpu.sync_copy(list_hbm.at[pl.ds(base + b * CH, CH)], wbuf)
        for k in range(8):
            v = wbuf[pl.ds(16 * k, 16)]
            idxb[pl.ds(16 * k, 16)] = (v & 16383) + t_off
            dlb[pl.ds(16 * k, 16)] = v >> 14
        _drain(table, idxb, dlb, gbuf, acc, sem)

    pltpu.sync_copy(acc.at[pl.ds(0, RPT)], out_hbm.at[pl.ds(out_off, RPT)])


_PROP_SCRATCH = [
    pltpu.VMEM((16,), jnp.int32),
    pltpu.VMEM((CH,), jnp.int32),
    pltpu.VMEM((CH,), jnp.int32),
    pltpu.VMEM((CH + 16,), jnp.int32),
    pltpu.VMEM((CH, CH), jnp.float32),
    pltpu.VMEM((AROWS, CH), jnp.float32),
    pltpu.SemaphoreType.DMA,
]


def _prop1_body(t_hbm, list_hbm, cnt_hbm, out_hbm,
                cv, wbuf, idxb, dlb, gbuf, acc, sem):
    c = lax.axis_index("c")
    s = lax.axis_index("s")
    _consume_phase(t_hbm, c * N_PAD, c * 16 + s, list_hbm, cnt_hbm,
                   c * N_PAD + s * RPT, out_hbm,
                   cv, wbuf, idxb, dlb, gbuf, acc, sem)


def _prop23_body(t1_hbm, t2_hbm, list_hbm, cnt_hbm, out_hbm,
                 cv, wbuf, idxb, dlb, gbuf, acc, sem):
    c = lax.axis_index("c")
    s = lax.axis_index("s")
    for g, tab in ((0, t1_hbm), (1, t2_hbm)):
        _consume_phase(tab, c * N_PAD, g * 16 + s, list_hbm, cnt_hbm,
                       (g * 2 + c) * N_PAD + s * RPT, out_hbm,
                       cv, wbuf, idxb, dlb, gbuf, acc, sem)


def _prop1(t_flat, lists, counts):
    k = pl.kernel(
        _prop1_body,
        out_type=jax.ShapeDtypeStruct((2 * N_PAD, CH), jnp.float32),
        mesh=_mesh(),
        scratch_types=_PROP_SCRATCH,
    )
    return k(t_flat, lists, counts).reshape(2, N_PAD, CH)


def _prop23(t1_flat, t2_flat, lists, counts):
    k = pl.kernel(
        _prop23_body,
        out_type=jax.ShapeDtypeStruct((4 * N_PAD, CH), jnp.float32),
        mesh=_mesh(),
        scratch_types=_PROP_SCRATCH,
    )
    return k(t1_flat, t2_flat, lists, counts).reshape(2, 2, N_PAD, CH)


def _pre_body(x_ref, dacc_ref, t_ref, dinv_ref):
    deg = dacc_ref[0, :, 0:1] + 1.0
    dv = lax.rsqrt(deg)
    dvb = jnp.broadcast_to(dv, (BLK, CH))
    dinv_ref[0] = dvb
    t_ref[0] = x_ref[0] * dvb


def _pre(x_pad, degacc):
    return pl.pallas_call(
        _pre_body,
        grid=(2, NBLK),
        in_specs=[
            pl.BlockSpec((1, BLK, CH), lambda g, i: (g, i, 0)),
            pl.BlockSpec((1, BLK, 16), lambda g, i: (g, i, 0)),
        ],
        out_specs=[
            pl.BlockSpec((1, BLK, CH), lambda g, i: (g, i, 0)),
            pl.BlockSpec((1, BLK, CH), lambda g, i: (g, i, 0)),
        ],
        out_shape=[
            jax.ShapeDtypeStruct((2, N_PAD, CH), jnp.float32),
            jax.ShapeDtypeStruct((2, N_PAD, CH), jnp.float32),
        ],
    )(x_pad, degacc)


def _layer_body(hin, scale_out, s_ref, t_ref, dinv_ref, w_ref, b_ref, out_ref):
    i = pl.program_id(1)
    dv = dinv_ref[0]
    acc = jnp.broadcast_to(b_ref[...], (BLK, 2 * CH))
    for hh in range(hin):
        u = (s_ref[0, hh] + t_ref[0, hh]) * dv
        acc = acc + jnp.dot(u, w_ref[hh], preferred_element_type=jnp.float32)
    rows = i * BLK + lax.broadcasted_iota(jnp.int32, (BLK, 2 * CH), 0)
    x = jnp.where(rows < N, jnp.maximum(acc, 0.0), 0.0)
    if scale_out:
        out_ref[0, 0] = x[:, :CH] * dv
        out_ref[0, 1] = x[:, CH:] * dv
    else:
        out_ref[0, 0] = x[:, :CH]
        out_ref[0, 1] = x[:, CH:]


def _layer(S, T, dinv, W, b, scale_out):
    hin = W.shape[0]
    return pl.pallas_call(
        functools.partial(_layer_body, hin, scale_out),
        grid=(2, NBLK),
        in_specs=[
            pl.BlockSpec((1, hin, BLK, CH), lambda g, i: (g, 0, i, 0)),
            pl.BlockSpec((1, hin, BLK, CH), lambda g, i: (g, 0, i, 0)),
            pl.BlockSpec((1, BLK, CH), lambda g, i: (g, i, 0)),
            pl.BlockSpec((hin, CH, 2 * CH), lambda g, i: (0, 0, 0)),
            pl.BlockSpec((1, 2 * CH), lambda g, i: (0, 0)),
        ],
        out_specs=pl.BlockSpec((1, 2, BLK, CH), lambda g, i: (g, 0, i, 0)),
        out_shape=jax.ShapeDtypeStruct((2, 2, N_PAD, CH), jnp.float32),
    )(S, T, dinv, W, b)


def _head_body(x1_ref, x2_ref, b1_ref, b2_ref,
               c1w, c1b, c2w, c2b, c3w, c3b, c4w, c4b, out_ref,
               z1a, z2a, cnt1, cnt2):
    i = pl.program_id(0)

    @pl.when(i == 0)
    def _():
        z1a[...] = jnp.zeros((G, 2 * CH), jnp.float32)
        z2a[...] = jnp.zeros((G, 2 * CH), jnp.float32)
        cnt1[...] = jnp.zeros((G, CH), jnp.float32)
        cnt2[...] = jnp.zeros((G, CH), jnp.float32)

    gid = lax.broadcasted_iota(jnp.int32, (G, BLK), 0)
    for x_ref, b_ref, za, ca in ((x1_ref, b1_ref, z1a, cnt1),
                                 (x2_ref, b2_ref, z2a, cnt2)):
        m = (jnp.broadcast_to(b_ref[...], (G, BLK)) == gid).astype(jnp.float32)
        h0 = jnp.dot(m, x_ref[0], preferred_element_type=jnp.float32)
        h1 = jnp.dot(m, x_ref[1], preferred_element_type=jnp.float32)
        za[...] += jnp.concatenate([h0, h1], axis=1)
        ca[...] += jnp.broadcast_to(
            jnp.sum(m, axis=1, keepdims=True), (G, CH))

    @pl.when(i == pl.num_programs(0) - 1)
    def _():
        z1 = z1a[...] / jnp.maximum(cnt1[...][:, 0:1], 1.0)
        z2 = z2a[...] / jnp.maximum(cnt2[...][:, 0:1], 1.0)
        z = jnp.concatenate([z1, z2], axis=1)
        z = jnp.maximum(jnp.dot(z, c1w[...], preferred_element_type=jnp.float32)
                        + c1b[...], 0.0)
        z = jnp.maximum(jnp.dot(z, c2w[...], preferred_element_type=jnp.float32)
                        + c2b[...], 0.0)
        z = jnp.maximum(jnp.dot(z, c3w[...], preferred_element_type=jnp.float32)
                        + c3b[...], 0.0)
        z = jnp.dot(z, c4w[...], preferred_element_type=jnp.float32) + c4b[...]
        out_ref[...] = jax.nn.sigmoid(z)


def _head(X3_1, X3_2, batch1, batch2, C1w, C1b, C2w, C2b, C3w, C3b, C4w, C4b):
    const2 = lambda a, b: pl.BlockSpec((a, b), lambda i: (0, 0))
    return pl.pallas_call(
        _head_body,
        grid=(NBLK,),
        in_specs=[
            pl.BlockSpec((2, BLK, CH), lambda i: (0, i, 0)),
            pl.BlockSpec((2, BLK, CH), lambda i: (0, i, 0)),
            pl.BlockSpec((1, BLK), lambda i: (0, i)),
            pl.BlockSpec((1, BLK), lambda i: (0, i)),
            const2(2 * H, 512), const2(1, 512),
            const2(512, 256), const2(1, 256),
            const2(256, 128), const2(1, 128),
            const2(128, 64), const2(1, 64),
        ],
        out_specs=pl.BlockSpec((G, G), lambda i: (0, 0)),
        out_shape=jax.ShapeDtypeStruct((G, G), jnp.float32),
        scratch_shapes=[
            pltpu.VMEM((G, 2 * CH), jnp.float32),
            pltpu.VMEM((G, 2 * CH), jnp.float32),
            pltpu.VMEM((G, CH), jnp.float32),
            pltpu.VMEM((G, CH), jnp.float32),
        ],
    )(X3_1, X3_2, batch1, batch2, C1w, C1b, C2w, C2b, C3w, C3b, C4w, C4b)


def _pack_edges(ei):
    ei = ei.astype(jnp.int32)
    w = ei[0] | (ei[1] << 14)
    return jnp.concatenate([w, jnp.full((E_PAD - E,), N_PAD << 14, jnp.int32)])


def kernel(x_1, edge_index_1, x_1_batch, x_2, edge_index_2, x_2_batch,
           W1, b1, W2, b2, W3, b3, C1w, C1b, C2w, C2b, C3w, C3b, C4w, C4b):
    w1 = _pack_edges(edge_index_1)
    w2 = _pack_edges(edge_index_2)

    x_pad = jnp.stack([
        jnp.pad(x_1, ((0, N_PAD - N), (0, 0))),
        jnp.pad(x_2, ((0, N_PAD - N), (0, 0))),
    ])

    degacc, lists, counts = _deg(w1, w2)
    T0, dinv = _pre(x_pad, degacc)

    S1 = _prop1(T0.reshape(2 * N_PAD, CH), lists, counts)
    T1 = _layer(S1.reshape(2, 1, N_PAD, CH), T0.reshape(2, 1, N_PAD, CH),
                dinv, W1.reshape(1, CH, 2 * CH), b1.reshape(1, 2 * CH), True)

    S2 = _prop23(T1[0].reshape(2 * N_PAD, CH), T1[1].reshape(2 * N_PAD, CH),
                 lists, counts)
    T2 = _layer(S2, T1, dinv, W2.reshape(2, CH, 2 * CH),
                b2.reshape(1, 2 * CH), True)

    S3 = _prop23(T2[0].reshape(2 * N_PAD, CH), T2[1].reshape(2 * N_PAD, CH),
                 lists, counts)
    X3 = _layer(S3, T2, dinv, W3.reshape(2, CH, 2 * CH),
                b3.reshape(1, 2 * CH), False)

    fillb = jnp.full((N_PAD - N,), G, jnp.int32)
    batch1 = jnp.concatenate([x_1_batch.astype(jnp.int32), fillb]).reshape(1, N_PAD)
    batch2 = jnp.concatenate([x_2_batch.astype(jnp.int32), fillb]).reshape(1, N_PAD)

    return _head(X3[0], X3[1], batch1, batch2,
                 C1w, C1b.reshape(1, -1), C2w, C2b.reshape(1, -1),
                 C3w, C3b.reshape(1, -1), C4w, C4b.reshape(1, -1))
